# baseline (speedup 1.0000x reference)
import functools

import jax
import jax.numpy as jnp
from jax import lax
from jax.experimental import pallas as pl
from jax.experimental.pallas import tpu as pltpu
from jax.experimental.pallas import tpu_sc as plsc

D = 1024
M = 4096
BATCH = 4
KSEL = 512
NQ = 16
NH = 16
DH = 64
MLPD = 4096
D4 = 256
AWR = 32
MB = 1024
IB = 512
JB = 1024
EPS = 1e-5

_SC_CORES = 2
_SC_SUBCORES = 16
_SC_WORKERS = _SC_CORES * _SC_SUBCORES
_ROWS_PER_W = (BATCH * KSEL) // _SC_WORKERS


def _gelu(x):
    return 0.5 * x * (1.0 + lax.erf(x * 0.7071067811865476))


def _ln(x, g, b):
    mu = jnp.mean(x, axis=-1, keepdims=True)
    xc = x - mu
    var = jnp.mean(xc * xc, axis=-1, keepdims=True)
    return xc / jnp.sqrt(var + EPS) * g + b


def _mlp1_body(tok_ref, w1t_ref, b1_ref, hid_ref):
    tok = tok_ref[0]
    hid_ref[0] = jnp.dot(tok.astype(jnp.bfloat16),
                         w1t_ref[...].astype(jnp.bfloat16),
                         preferred_element_type=jnp.float32) + b1_ref[...]


def _mlp1(tokens, w1t, b1):
    return pl.pallas_call(
        _mlp1_body,
        grid=(BATCH, M // MB),
        in_specs=[
            pl.BlockSpec((1, MB, D), lambda b, m: (b, m, 0)),
            pl.BlockSpec((D, D4), lambda b, m: (0, 0)),
            pl.BlockSpec((1, D4), lambda b, m: (0, 0)),
        ],
        out_specs=pl.BlockSpec((1, MB, D4), lambda b, m: (b, m, 0)),
        out_shape=jax.ShapeDtypeStruct((BATCH, M, D4), jnp.float32),
    )(tokens, w1t, b1)


def _imp_body(hid_ref, aw_ref, maskf_ref, w2t_ref, b2_ref, imp_ref, comp_ref):
    hid = hid_ref[0]
    logit = jnp.dot(hid.astype(jnp.bfloat16), w2t_ref[...].astype(jnp.bfloat16),
                    preferred_element_type=jnp.float32) + b2_ref[...]
    imp = jax.nn.sigmoid(logit)
    cov = jnp.max(aw_ref[0], axis=1, keepdims=True)
    comp = imp * (1.0 - cov)
    comp = jnp.where(maskf_ref[0] > 0.0, comp, -jnp.inf)
    imp_ref[0] = imp
    comp_ref[0] = comp


def _importance(hid, aw_c, maskf, w2t, b2):
    grid = (BATCH, M // MB)
    return pl.pallas_call(
        _imp_body,
        grid=grid,
        in_specs=[
            pl.BlockSpec((1, MB, D4), lambda b, m: (b, m, 0)),
            pl.BlockSpec((1, MB, AWR), lambda b, m: (b, m, 0)),
            pl.BlockSpec((1, MB, 1), lambda b, m: (b, m, 0)),
            pl.BlockSpec((D4, 1), lambda b, m: (0, 0)),
            pl.BlockSpec((1, 1), lambda b, m: (0, 0)),
        ],
        out_specs=[
            pl.BlockSpec((1, MB, 1), lambda b, m: (b, m, 0)),
            pl.BlockSpec((1, MB, 1), lambda b, m: (b, m, 0)),
        ],
        out_shape=[
            jax.ShapeDtypeStruct((BATCH, M, 1), jnp.float32),
            jax.ShapeDtypeStruct((BATCH, M, 1), jnp.float32),
        ],
    )(hid, aw_c, maskf, w2t, b2)


def _topk_body(compr_ref, compc_ref, sel_ref, gidx_ref):
    b = pl.program_id(0)
    vr = compr_ref[0]
    vc = compc_ref[0]
    jj = lax.broadcasted_iota(jnp.int32, (1, M), 1)
    pp = lax.broadcasted_iota(jnp.int32, (1, KSEL), 1)
    sel_acc = jnp.zeros((1, KSEL), jnp.int32)
    for ib in range(M // IB):
        vi = vc[ib * IB:(ib + 1) * IB, :]
        ii = lax.broadcasted_iota(jnp.int32, (IB, 1), 0) + ib * IB
        better = jnp.logical_or(
            vr > vi, jnp.logical_and(vr == vi, jj < ii))
        cnt = jnp.sum(better.astype(jnp.int32), axis=1, keepdims=True)
        match = cnt == pp
        sel_acc = sel_acc + jnp.sum(
            jnp.where(match, ii, 0), axis=0, keepdims=True)
    sel_ref[0] = sel_acc
    gidx_ref[0] = sel_acc + b * M


def _topk(compr, compc):
    return pl.pallas_call(
        _topk_body,
        grid=(BATCH,),
        in_specs=[
            pl.BlockSpec((1, 1, M), lambda b: (b, 0, 0)),
            pl.BlockSpec((1, M, 1), lambda b: (b, 0, 0)),
        ],
        out_specs=[
            pl.BlockSpec((1, 1, KSEL), lambda b: (b, 0, 0)),
            pl.BlockSpec((1, 1, KSEL), lambda b: (b, 0, 0)),
        ],
        out_shape=[
            jax.ShapeDtypeStruct((BATCH, 1, KSEL), jnp.int32),
            jax.ShapeDtypeStruct((BATCH, 1, KSEL), jnp.int32),
        ],
    )(compr, compc)


def _sc_gather(tokens_flat, gidx):
    mesh = plsc.VectorSubcoreMesh(core_axis_name="c", subcore_axis_name="s")

    @functools.partial(
        pl.kernel,
        mesh=mesh,
        out_type=jax.ShapeDtypeStruct((BATCH * KSEL, D), jnp.float32),
        scratch_types=[
            pltpu.VMEM((_ROWS_PER_W,), jnp.int32),
            pltpu.VMEM((_ROWS_PER_W, D), jnp.float32),
            pltpu.SemaphoreType.DMA,
        ],
    )
    def k(tok_hbm, gidx_hbm, out_hbm, idx_v, rows_v, sem):
        wid = lax.axis_index("s") * _SC_CORES + lax.axis_index("c")
        base = wid * _ROWS_PER_W
        pltpu.sync_copy(gidx_hbm.at[pl.ds(base, _ROWS_PER_W)], idx_v)
        pltpu.async_copy(tok_hbm.at[idx_v], rows_v, sem).wait()
        pltpu.sync_copy(rows_v, out_hbm.at[pl.ds(base, _ROWS_PER_W)])

    return k(tokens_flat, gidx)


def _attn_body(q_ref, kv_ref, vecs_ref, wqt_ref, wkt_ref, wvt_ref, wot_ref,
               q1_ref):
    q = q_ref[0]
    kv = kv_ref[0]
    vecs = vecs_ref[...]
    qn = _ln(q, vecs[0:1], vecs[1:2])
    kvn = _ln(kv, vecs[2:3], vecs[3:4])
    qp = jnp.dot(qn, wqt_ref[...], preferred_element_type=jnp.float32) + vecs[4:5]
    kp = jnp.dot(kvn, wkt_ref[...], preferred_element_type=jnp.float32) + vecs[5:6]
    vp = jnp.dot(kv, wvt_ref[...], preferred_element_type=jnp.float32) + vecs[6:7]
    outs = []
    for h in range(NH):
        sl = slice(h * DH, (h + 1) * DH)
        qh = qp[:, sl]
        kh = kp[:, sl]
        vh = vp[:, sl]
        sc = lax.dot_general(qh, kh, (((1,), (1,)), ((), ())),
                             preferred_element_type=jnp.float32) * 0.125
        m = jnp.max(sc, axis=1, keepdims=True)
        e = jnp.exp(sc - m)
        a = e / jnp.sum(e, axis=1, keepdims=True)
        outs.append(jnp.dot(a, vh, preferred_element_type=jnp.float32))
    ao = jnp.concatenate(outs, axis=1)
    o = jnp.dot(ao, wot_ref[...], preferred_element_type=jnp.float32) + vecs[7:8]
    q1_ref[0] = q + o


def _attn(q, kv, vecs, wqt, wkt, wvt, wot):
    return pl.pallas_call(
        _attn_body,
        grid=(BATCH,),
        in_specs=[
            pl.BlockSpec((1, NQ, D), lambda b: (b, 0, 0)),
            pl.BlockSpec((1, KSEL, D), lambda b: (b, 0, 0)),
            pl.BlockSpec((8, D), lambda b: (0, 0)),
            pl.BlockSpec((D, D), lambda b: (0, 0)),
            pl.BlockSpec((D, D), lambda b: (0, 0)),
            pl.BlockSpec((D, D), lambda b: (0, 0)),
            pl.BlockSpec((D, D), lambda b: (0, 0)),
        ],
        out_specs=pl.BlockSpec((1, NQ, D), lambda b: (b, 0, 0)),
        out_shape=jax.ShapeDtypeStruct((BATCH, NQ, D), jnp.float32),
    )(q, kv, vecs, wqt, wkt, wvt, wot)


def _ffn_body(x_ref, lnv_ref, f1t_ref, f1b_ref, f2r_ref, f2b_ref, out_ref,
              hm_s, acc_s, *, final_ln):
    j = pl.program_id(0)

    @pl.when(j == 0)
    def _():
        lnv = lnv_ref[...]
        hm_s[...] = _ln(x_ref[...], lnv[0:1], lnv[1:2])
        acc_s[...] = jnp.zeros_like(acc_s)

    h = jnp.dot(hm_s[...], f1t_ref[...], preferred_element_type=jnp.float32)
    h = _gelu(h + f1b_ref[...])
    acc_s[...] += jnp.dot(h, f2r_ref[...], preferred_element_type=jnp.float32)

    @pl.when(j == MLPD // JB - 1)
    def _():
        out = x_ref[...] + acc_s[...] + f2b_ref[...]
        if final_ln:
            lnv = lnv_ref[...]
            out = _ln(out, lnv[2:3], lnv[3:4])
        out_ref[...] = out


def _ffn(x, lnv, f1t, f1b, f2r, f2b, final_ln):
    bq = BATCH * NQ
    return pl.pallas_call(
        functools.partial(_ffn_body, final_ln=final_ln),
        grid=(MLPD // JB,),
        in_specs=[
            pl.BlockSpec((bq, D), lambda j: (0, 0)),
            pl.BlockSpec((4, D), lambda j: (0, 0)),
            pl.BlockSpec((D, JB), lambda j: (0, j)),
            pl.BlockSpec((1, JB), lambda j: (0, j)),
            pl.BlockSpec((JB, D), lambda j: (j, 0)),
            pl.BlockSpec((1, D), lambda j: (0, 0)),
        ],
        out_specs=pl.BlockSpec((bq, D), lambda j: (0, 0)),
        out_shape=jax.ShapeDtypeStruct((bq, D), jnp.float32),
        scratch_shapes=[
            pltpu.VMEM((bq, D), jnp.float32),
            pltpu.VMEM((bq, D), jnp.float32),
        ],
    )(x, lnv, f1t, f1b, f2r, f2b)


def kernel(tokens, attention_weights, mask, params):
    aw_c = attention_weights.transpose(1, 3, 0, 2).reshape(BATCH, M, AWR)
    maskf = mask.astype(jnp.float32)[..., None]
    w1t = params['imp_w1'].T
    b1 = params['imp_b1'][None]
    w2t = params['imp_w2'].T
    b2 = params['imp_b2'][None]

    hid_pre = _mlp1(tokens, w1t, b1)
    hid = jax.nn.gelu(hid_pre, approximate=False)
    imp3, comp3 = _importance(hid, aw_c, maskf, w2t, b2)
    importance = imp3[..., 0]

    compr = comp3.reshape(BATCH, 1, M)
    sel3, gidx3 = _topk(compr, comp3)
    selected_indices = sel3[:, 0, :]

    sel_tokens_flat = _sc_gather(tokens.reshape(BATCH * M, D),
                                 gidx3.reshape(BATCH * KSEL))
    sel_tokens = sel_tokens_flat.reshape(BATCH, KSEL, D)

    q = jnp.broadcast_to(
        (params['detail_queries'] + params['detail_pos'])[None],
        (BATCH, NQ, D))

    n_layers = len(params['layers'])
    for li, p in enumerate(params['layers']):
        wq, wk, wv = jnp.split(p['in_w'], 3, axis=0)
        bq_, bk_, bv_ = jnp.split(p['in_b'], 3)
        vecs = jnp.stack([p['nq_g'], p['nq_b'], p['nkv_g'], p['nkv_b'],
                          bq_, bk_, bv_, p['out_b']])
        q = _attn(q, sel_tokens, vecs, wq.T, wk.T, wv.T, p['out_w'].T)
        lnv = jnp.stack([p['nffn_g'], p['nffn_b'],
                         params['on_g'], params['on_b']])
        x = q.reshape(BATCH * NQ, D)
        x = _ffn(x, lnv, p['f1_w'].T, p['f1_b'][None], p['f2_w'].T,
                 p['f2_b'][None], final_ln=(li == n_layers - 1))
        q = x.reshape(BATCH, NQ, D)

    return q, importance, selected_indices


# speedup vs baseline: 1.3233x; 1.3233x over previous
import functools

import jax
import jax.numpy as jnp
from jax import lax
from jax.experimental import pallas as pl
from jax.experimental.pallas import tpu as pltpu
from jax.experimental.pallas import tpu_sc as plsc

D = 1024
M = 4096
BATCH = 4
KSEL = 512
NQ = 16
NH = 16
DH = 64
MLPD = 4096
D4 = 256
AWR = 32
MB = 1024
IB = 512
JB = 1024
EPS = 1e-5

_SC_CORES = 2
_SC_SUBCORES = 16
_SC_WORKERS = _SC_CORES * _SC_SUBCORES
_ROWS_PER_W = (BATCH * KSEL) // _SC_WORKERS


def _gelu(x):
    return 0.5 * x * (1.0 + lax.erf(x * 0.7071067811865476))


def _ln(x, g, b):
    mu = jnp.mean(x, axis=-1, keepdims=True)
    xc = x - mu
    var = jnp.mean(xc * xc, axis=-1, keepdims=True)
    return xc / jnp.sqrt(var + EPS) * g + b


def _mlp1_body(tok_ref, w1t_ref, b1_ref, hid_ref):
    tok = tok_ref[0]
    hid_ref[0] = jnp.dot(tok.astype(jnp.bfloat16),
                         w1t_ref[...].astype(jnp.bfloat16),
                         preferred_element_type=jnp.float32) + b1_ref[...]


def _mlp1(tokens, w1t, b1):
    return pl.pallas_call(
        _mlp1_body,
        grid=(BATCH, M // MB),
        in_specs=[
            pl.BlockSpec((1, MB, D), lambda b, m: (b, m, 0)),
            pl.BlockSpec((D, D4), lambda b, m: (0, 0)),
            pl.BlockSpec((1, D4), lambda b, m: (0, 0)),
        ],
        out_specs=pl.BlockSpec((1, MB, D4), lambda b, m: (b, m, 0)),
        out_shape=jax.ShapeDtypeStruct((BATCH, M, D4), jnp.float32),
    )(tokens, w1t, b1)


def _imp_body(hid_ref, aw_ref, maskf_ref, w2t_ref, b2_ref, imp_ref, comp_ref):
    hid = hid_ref[0]
    logit = jnp.dot(hid.astype(jnp.bfloat16), w2t_ref[...].astype(jnp.bfloat16),
                    preferred_element_type=jnp.float32) + b2_ref[...]
    imp = jax.nn.sigmoid(logit)
    cov = jnp.max(aw_ref[0], axis=1, keepdims=True)
    comp = imp * (1.0 - cov)
    comp = jnp.where(maskf_ref[0] > 0.0, comp, -jnp.inf)
    imp_ref[0] = imp
    comp_ref[0] = comp


def _importance(hid, aw_c, maskf, w2t, b2):
    grid = (BATCH, M // MB)
    return pl.pallas_call(
        _imp_body,
        grid=grid,
        in_specs=[
            pl.BlockSpec((1, MB, D4), lambda b, m: (b, m, 0)),
            pl.BlockSpec((1, MB, AWR), lambda b, m: (b, m, 0)),
            pl.BlockSpec((1, MB, 1), lambda b, m: (b, m, 0)),
            pl.BlockSpec((D4, 1), lambda b, m: (0, 0)),
            pl.BlockSpec((1, 1), lambda b, m: (0, 0)),
        ],
        out_specs=[
            pl.BlockSpec((1, MB, 1), lambda b, m: (b, m, 0)),
            pl.BlockSpec((1, MB, 1), lambda b, m: (b, m, 0)),
        ],
        out_shape=[
            jax.ShapeDtypeStruct((BATCH, M, 1), jnp.float32),
            jax.ShapeDtypeStruct((BATCH, M, 1), jnp.float32),
        ],
    )(hid, aw_c, maskf, w2t, b2)


def _topk_body(compr_ref, compc_ref, sel_ref, gidx_ref):
    b = pl.program_id(0)
    vr = compr_ref[0]
    vc = compc_ref[0]
    jj = lax.broadcasted_iota(jnp.int32, (1, M), 1)
    pp = lax.broadcasted_iota(jnp.int32, (1, KSEL), 1)
    sel_acc = jnp.zeros((1, KSEL), jnp.int32)
    for ib in range(M // IB):
        vi = vc[ib * IB:(ib + 1) * IB, :]
        ii = lax.broadcasted_iota(jnp.int32, (IB, 1), 0) + ib * IB
        better = jnp.logical_or(
            vr > vi, jnp.logical_and(vr == vi, jj < ii))
        cnt = jnp.sum(better.astype(jnp.int32), axis=1, keepdims=True)
        match = cnt == pp
        sel_acc = sel_acc + jnp.sum(
            jnp.where(match, ii, 0), axis=0, keepdims=True)
    sel_ref[0] = sel_acc
    gidx_ref[0] = sel_acc + b * M


def _topk(compr, compc):
    return pl.pallas_call(
        _topk_body,
        grid=(BATCH,),
        in_specs=[
            pl.BlockSpec((1, 1, M), lambda b: (b, 0, 0)),
            pl.BlockSpec((1, M, 1), lambda b: (b, 0, 0)),
        ],
        out_specs=[
            pl.BlockSpec((1, 1, KSEL), lambda b: (b, 0, 0)),
            pl.BlockSpec((1, 1, KSEL), lambda b: (b, 0, 0)),
        ],
        out_shape=[
            jax.ShapeDtypeStruct((BATCH, 1, KSEL), jnp.int32),
            jax.ShapeDtypeStruct((BATCH, 1, KSEL), jnp.int32),
        ],
    )(compr, compc)


def _sc_gather(tokens_flat, gidx):
    mesh = plsc.VectorSubcoreMesh(core_axis_name="c", subcore_axis_name="s")

    @functools.partial(
        pl.kernel,
        mesh=mesh,
        out_type=jax.ShapeDtypeStruct((BATCH * KSEL, D), jnp.float32),
        scratch_types=[
            pltpu.VMEM((_ROWS_PER_W,), jnp.int32),
            pltpu.VMEM((_ROWS_PER_W, D), jnp.float32),
            pltpu.SemaphoreType.DMA,
        ],
    )
    def k(tok_hbm, gidx_hbm, out_hbm, idx_v, rows_v, sem):
        wid = lax.axis_index("s") * _SC_CORES + lax.axis_index("c")
        base = wid * _ROWS_PER_W
        pltpu.sync_copy(gidx_hbm.at[pl.ds(base, _ROWS_PER_W)], idx_v)
        pltpu.async_copy(tok_hbm.at[idx_v], rows_v, sem).wait()
        pltpu.sync_copy(rows_v, out_hbm.at[pl.ds(base, _ROWS_PER_W)])

    return k(tokens_flat, gidx)


_NT = (((1,), (1,)), ((), ()))


def _attn_body(q_ref, kv_ref, vecs_ref, inw_ref, wo_ref, q1_ref):
    q = q_ref[0]
    kv = kv_ref[0]
    vecs = vecs_ref[...]
    qn = _ln(q, vecs[0:1], vecs[1:2])
    kvn = _ln(kv, vecs[2:3], vecs[3:4])
    wq = inw_ref[0:D, :]
    wk = inw_ref[D:2 * D, :]
    wv = inw_ref[2 * D:3 * D, :]
    qp = lax.dot_general(qn, wq, _NT,
                         preferred_element_type=jnp.float32) + vecs[4:5]
    kp = lax.dot_general(kvn, wk, _NT,
                         preferred_element_type=jnp.float32) + vecs[5:6]
    vp = lax.dot_general(kv, wv, _NT,
                         preferred_element_type=jnp.float32) + vecs[6:7]
    outs = []
    for h in range(NH):
        sl = slice(h * DH, (h + 1) * DH)
        qh = qp[:, sl]
        kh = kp[:, sl]
        vh = vp[:, sl]
        sc = lax.dot_general(qh, kh, (((1,), (1,)), ((), ())),
                             preferred_element_type=jnp.float32) * 0.125
        m = jnp.max(sc, axis=1, keepdims=True)
        e = jnp.exp(sc - m)
        a = e / jnp.sum(e, axis=1, keepdims=True)
        outs.append(jnp.dot(a, vh, preferred_element_type=jnp.float32))
    ao = jnp.concatenate(outs, axis=1)
    o = lax.dot_general(ao, wo_ref[...], _NT,
                        preferred_element_type=jnp.float32) + vecs[7:8]
    q1_ref[0] = q + o


def _attn(q, kv, vecs, inw, wo):
    return pl.pallas_call(
        _attn_body,
        grid=(BATCH,),
        in_specs=[
            pl.BlockSpec((1, NQ, D), lambda b: (b, 0, 0)),
            pl.BlockSpec((1, KSEL, D), lambda b: (b, 0, 0)),
            pl.BlockSpec((8, D), lambda b: (0, 0)),
            pl.BlockSpec((3 * D, D), lambda b: (0, 0)),
            pl.BlockSpec((D, D), lambda b: (0, 0)),
        ],
        out_specs=pl.BlockSpec((1, NQ, D), lambda b: (b, 0, 0)),
        out_shape=jax.ShapeDtypeStruct((BATCH, NQ, D), jnp.float32),
    )(q, kv, vecs, inw, wo)


def _ffn_body(x_ref, lnv_ref, f1_ref, f1b_ref, f2_ref, f2b_ref, out_ref,
              hm_s, acc_s, *, final_ln):
    j = pl.program_id(0)

    @pl.when(j == 0)
    def _():
        lnv = lnv_ref[...]
        hm_s[...] = _ln(x_ref[...], lnv[0:1], lnv[1:2])
        acc_s[...] = jnp.zeros_like(acc_s)

    h = lax.dot_general(hm_s[...], f1_ref[...], _NT,
                        preferred_element_type=jnp.float32)
    h = _gelu(h + f1b_ref[...])
    acc_s[...] += lax.dot_general(h, f2_ref[...], _NT,
                                  preferred_element_type=jnp.float32)

    @pl.when(j == MLPD // JB - 1)
    def _():
        out = x_ref[...] + acc_s[...] + f2b_ref[...]
        if final_ln:
            lnv = lnv_ref[...]
            out = _ln(out, lnv[2:3], lnv[3:4])
        out_ref[...] = out


def _ffn(x, lnv, f1, f1b, f2, f2b, final_ln):
    bq = BATCH * NQ
    return pl.pallas_call(
        functools.partial(_ffn_body, final_ln=final_ln),
        grid=(MLPD // JB,),
        in_specs=[
            pl.BlockSpec((bq, D), lambda j: (0, 0)),
            pl.BlockSpec((4, D), lambda j: (0, 0)),
            pl.BlockSpec((JB, D), lambda j: (j, 0)),
            pl.BlockSpec((1, JB), lambda j: (0, j)),
            pl.BlockSpec((D, JB), lambda j: (0, j)),
            pl.BlockSpec((1, D), lambda j: (0, 0)),
        ],
        out_specs=pl.BlockSpec((bq, D), lambda j: (0, 0)),
        out_shape=jax.ShapeDtypeStruct((bq, D), jnp.float32),
        scratch_shapes=[
            pltpu.VMEM((bq, D), jnp.float32),
            pltpu.VMEM((bq, D), jnp.float32),
        ],
    )(x, lnv, f1, f1b, f2, f2b)


def kernel(tokens, attention_weights, mask, params):
    aw_c = attention_weights.transpose(1, 3, 0, 2).reshape(BATCH, M, AWR)
    maskf = mask.astype(jnp.float32)[..., None]
    w1t = params['imp_w1'].T
    b1 = params['imp_b1'][None]
    w2t = params['imp_w2'].T
    b2 = params['imp_b2'][None]

    hid_pre = _mlp1(tokens, w1t, b1)
    hid = jax.nn.gelu(hid_pre, approximate=False)
    imp3, comp3 = _importance(hid, aw_c, maskf, w2t, b2)
    importance = imp3[..., 0]

    compr = comp3.reshape(BATCH, 1, M)
    sel3, gidx3 = _topk(compr, comp3)
    selected_indices = sel3[:, 0, :]

    sel_tokens_flat = _sc_gather(tokens.reshape(BATCH * M, D),
                                 gidx3.reshape(BATCH * KSEL))
    sel_tokens = sel_tokens_flat.reshape(BATCH, KSEL, D)

    q = jnp.broadcast_to(
        (params['detail_queries'] + params['detail_pos'])[None],
        (BATCH, NQ, D))

    n_layers = len(params['layers'])
    for li, p in enumerate(params['layers']):
        bq_, bk_, bv_ = jnp.split(p['in_b'], 3)
        vecs = jnp.stack([p['nq_g'], p['nq_b'], p['nkv_g'], p['nkv_b'],
                          bq_, bk_, bv_, p['out_b']])
        q = _attn(q, sel_tokens, vecs, p['in_w'], p['out_w'])
        lnv = jnp.stack([p['nffn_g'], p['nffn_b'],
                         params['on_g'], params['on_b']])
        x = q.reshape(BATCH * NQ, D)
        x = _ffn(x, lnv, p['f1_w'], p['f1_b'][None], p['f2_w'],
                 p['f2_b'][None], final_ln=(li == n_layers - 1))
        q = x.reshape(BATCH, NQ, D)

    return q, importance, selected_indices


# speedup vs baseline: 1.3245x; 1.0009x over previous
import functools

import jax
import jax.numpy as jnp
from jax import lax
from jax.experimental import pallas as pl
from jax.experimental.pallas import tpu as pltpu
from jax.experimental.pallas import tpu_sc as plsc

D = 1024
M = 4096
BATCH = 4
KSEL = 512
NQ = 16
NH = 16
DH = 64
MLPD = 4096
D4 = 256
AWR = 32
MB = 1024
IB = 512
JB = 1024
EPS = 1e-5

_SC_CORES = 2
_SC_SUBCORES = 16
_SC_WORKERS = _SC_CORES * _SC_SUBCORES
_ROWS_PER_W = (BATCH * KSEL) // _SC_WORKERS


def _gelu(x):
    return 0.5 * x * (1.0 + lax.erf(x * 0.7071067811865476))


def _ln(x, g, b):
    mu = jnp.mean(x, axis=-1, keepdims=True)
    xc = x - mu
    var = jnp.mean(xc * xc, axis=-1, keepdims=True)
    return xc / jnp.sqrt(var + EPS) * g + b


def _mlp1_body(tok_ref, w1t_ref, b1_ref, hid_ref):
    tok = tok_ref[0]
    hid_ref[0] = jnp.dot(tok.astype(jnp.bfloat16),
                         w1t_ref[...].astype(jnp.bfloat16),
                         preferred_element_type=jnp.float32) + b1_ref[...]


def _mlp1(tokens, w1t, b1):
    return pl.pallas_call(
        _mlp1_body,
        grid=(BATCH, M // MB),
        in_specs=[
            pl.BlockSpec((1, MB, D), lambda b, m: (b, m, 0)),
            pl.BlockSpec((D, D4), lambda b, m: (0, 0)),
            pl.BlockSpec((1, D4), lambda b, m: (0, 0)),
        ],
        out_specs=pl.BlockSpec((1, MB, D4), lambda b, m: (b, m, 0)),
        out_shape=jax.ShapeDtypeStruct((BATCH, M, D4), jnp.float32),
    )(tokens, w1t, b1)


def _imp_body(hid_ref, aw_ref, maskf_ref, w2t_ref, b2_ref, imp_ref, comp_ref):
    hid = hid_ref[0]
    logit = jnp.dot(hid.astype(jnp.bfloat16), w2t_ref[...].astype(jnp.bfloat16),
                    preferred_element_type=jnp.float32) + b2_ref[...]
    imp = jax.nn.sigmoid(logit)
    cov = jnp.max(aw_ref[0], axis=1, keepdims=True)
    comp = imp * (1.0 - cov)
    comp = jnp.where(maskf_ref[0] > 0.0, comp, -jnp.inf)
    imp_ref[0] = imp
    comp_ref[0] = comp


def _importance(hid, aw_c, maskf, w2t, b2):
    grid = (BATCH, M // MB)
    return pl.pallas_call(
        _imp_body,
        grid=grid,
        in_specs=[
            pl.BlockSpec((1, MB, D4), lambda b, m: (b, m, 0)),
            pl.BlockSpec((1, MB, AWR), lambda b, m: (b, m, 0)),
            pl.BlockSpec((1, MB, 1), lambda b, m: (b, m, 0)),
            pl.BlockSpec((D4, 1), lambda b, m: (0, 0)),
            pl.BlockSpec((1, 1), lambda b, m: (0, 0)),
        ],
        out_specs=[
            pl.BlockSpec((1, MB, 1), lambda b, m: (b, m, 0)),
            pl.BlockSpec((1, MB, 1), lambda b, m: (b, m, 0)),
        ],
        out_shape=[
            jax.ShapeDtypeStruct((BATCH, M, 1), jnp.float32),
            jax.ShapeDtypeStruct((BATCH, M, 1), jnp.float32),
        ],
    )(hid, aw_c, maskf, w2t, b2)


def _topk_body(compr_ref, compc_ref, sel_ref, gidx_ref):
    b = pl.program_id(0)
    vr = compr_ref[0]
    vc = compc_ref[0]
    jj = lax.broadcasted_iota(jnp.int32, (1, M), 1)
    pp = lax.broadcasted_iota(jnp.int32, (1, KSEL), 1)
    sel_acc = jnp.zeros((1, KSEL), jnp.int32)
    for ib in range(M // IB):
        vi = vc[ib * IB:(ib + 1) * IB, :]
        ii = lax.broadcasted_iota(jnp.int32, (IB, 1), 0) + ib * IB
        better = jnp.logical_or(
            vr > vi, jnp.logical_and(vr == vi, jj < ii))
        cnt = jnp.sum(better.astype(jnp.int32), axis=1, keepdims=True)
        match = cnt == pp
        sel_acc = sel_acc + jnp.sum(
            jnp.where(match, ii, 0), axis=0, keepdims=True)
    sel_ref[0] = sel_acc
    gidx_ref[0] = sel_acc + b * M


def _topk(compr, compc):
    return pl.pallas_call(
        _topk_body,
        grid=(BATCH,),
        in_specs=[
            pl.BlockSpec((1, 1, M), lambda b: (b, 0, 0)),
            pl.BlockSpec((1, M, 1), lambda b: (b, 0, 0)),
        ],
        out_specs=[
            pl.BlockSpec((1, 1, KSEL), lambda b: (b, 0, 0)),
            pl.BlockSpec((1, 1, KSEL), lambda b: (b, 0, 0)),
        ],
        out_shape=[
            jax.ShapeDtypeStruct((BATCH, 1, KSEL), jnp.int32),
            jax.ShapeDtypeStruct((BATCH, 1, KSEL), jnp.int32),
        ],
    )(compr, compc)


def _sc_gather(tokens_flat, gidx):
    mesh = plsc.VectorSubcoreMesh(core_axis_name="c", subcore_axis_name="s")

    @functools.partial(
        pl.kernel,
        mesh=mesh,
        out_type=jax.ShapeDtypeStruct((BATCH * KSEL, D), jnp.float32),
        scratch_types=[
            pltpu.VMEM((_ROWS_PER_W,), jnp.int32),
            pltpu.VMEM((_ROWS_PER_W, D), jnp.float32),
            pltpu.SemaphoreType.DMA,
        ],
    )
    def k(tok_hbm, gidx_hbm, out_hbm, idx_v, rows_v, sem):
        wid = lax.axis_index("s") * _SC_CORES + lax.axis_index("c")
        base = wid * _ROWS_PER_W
        pltpu.sync_copy(gidx_hbm.at[pl.ds(base, _ROWS_PER_W)], idx_v)
        pltpu.async_copy(tok_hbm.at[idx_v], rows_v, sem).wait()
        pltpu.sync_copy(rows_v, out_hbm.at[pl.ds(base, _ROWS_PER_W)])

    return k(tokens_flat, gidx)


_NT = (((1,), (1,)), ((), ()))


def _attn_body(q_ref, kv_ref, vecs_ref, inw_ref, wo_ref, q1_ref):
    q = q_ref[0]
    kv = kv_ref[0]
    vecs = vecs_ref[...]
    qn = _ln(q, vecs[0:1], vecs[1:2])
    kvn = _ln(kv, vecs[2:3], vecs[3:4])
    wq = inw_ref[0:D, :]
    wk = inw_ref[D:2 * D, :]
    wv = inw_ref[2 * D:3 * D, :]
    bf = jnp.bfloat16
    qp = lax.dot_general(qn.astype(bf), wq.astype(bf), _NT,
                         preferred_element_type=jnp.float32) + vecs[4:5]
    kp = lax.dot_general(kvn.astype(bf), wk.astype(bf), _NT,
                         preferred_element_type=jnp.float32) + vecs[5:6]
    vp = lax.dot_general(kv.astype(bf), wv.astype(bf), _NT,
                         preferred_element_type=jnp.float32) + vecs[6:7]
    outs = []
    for h in range(NH):
        sl = slice(h * DH, (h + 1) * DH)
        qh = qp[:, sl]
        kh = kp[:, sl]
        vh = vp[:, sl]
        sc = lax.dot_general(qh.astype(bf), kh.astype(bf), _NT,
                             preferred_element_type=jnp.float32) * 0.125
        m = jnp.max(sc, axis=1, keepdims=True)
        e = jnp.exp(sc - m)
        a = e / jnp.sum(e, axis=1, keepdims=True)
        outs.append(lax.dot_general(a.astype(bf), vh.astype(bf), (((1,), (0,)), ((), ())),
                                    preferred_element_type=jnp.float32))
    ao = jnp.concatenate(outs, axis=1)
    o = lax.dot_general(ao.astype(bf), wo_ref[...].astype(bf), _NT,
                        preferred_element_type=jnp.float32) + vecs[7:8]
    q1_ref[0] = q + o


def _attn(q, kv, vecs, inw, wo):
    return pl.pallas_call(
        _attn_body,
        grid=(BATCH,),
        in_specs=[
            pl.BlockSpec((1, NQ, D), lambda b: (b, 0, 0)),
            pl.BlockSpec((1, KSEL, D), lambda b: (b, 0, 0)),
            pl.BlockSpec((8, D), lambda b: (0, 0)),
            pl.BlockSpec((3 * D, D), lambda b: (0, 0)),
            pl.BlockSpec((D, D), lambda b: (0, 0)),
        ],
        out_specs=pl.BlockSpec((1, NQ, D), lambda b: (b, 0, 0)),
        out_shape=jax.ShapeDtypeStruct((BATCH, NQ, D), jnp.float32),
    )(q, kv, vecs, inw, wo)


def _ffn_body(x_ref, lnv_ref, f1_ref, f1b_ref, f2_ref, f2b_ref, out_ref,
              hm_s, acc_s, *, final_ln):
    j = pl.program_id(0)

    @pl.when(j == 0)
    def _():
        lnv = lnv_ref[...]
        hm_s[...] = _ln(x_ref[...], lnv[0:1], lnv[1:2])
        acc_s[...] = jnp.zeros_like(acc_s)

    h = lax.dot_general(hm_s[...].astype(jnp.bfloat16),
                        f1_ref[...].astype(jnp.bfloat16), _NT,
                        preferred_element_type=jnp.float32)
    h = _gelu(h + f1b_ref[...])
    acc_s[...] += lax.dot_general(h.astype(jnp.bfloat16),
                                  f2_ref[...].astype(jnp.bfloat16), _NT,
                                  preferred_element_type=jnp.float32)

    @pl.when(j == MLPD // JB - 1)
    def _():
        out = x_ref[...] + acc_s[...] + f2b_ref[...]
        if final_ln:
            lnv = lnv_ref[...]
            out = _ln(out, lnv[2:3], lnv[3:4])
        out_ref[...] = out


def _ffn(x, lnv, f1, f1b, f2, f2b, final_ln):
    bq = BATCH * NQ
    return pl.pallas_call(
        functools.partial(_ffn_body, final_ln=final_ln),
        grid=(MLPD // JB,),
        in_specs=[
            pl.BlockSpec((bq, D), lambda j: (0, 0)),
            pl.BlockSpec((4, D), lambda j: (0, 0)),
            pl.BlockSpec((JB, D), lambda j: (j, 0)),
            pl.BlockSpec((1, JB), lambda j: (0, j)),
            pl.BlockSpec((D, JB), lambda j: (0, j)),
            pl.BlockSpec((1, D), lambda j: (0, 0)),
        ],
        out_specs=pl.BlockSpec((bq, D), lambda j: (0, 0)),
        out_shape=jax.ShapeDtypeStruct((bq, D), jnp.float32),
        scratch_shapes=[
            pltpu.VMEM((bq, D), jnp.float32),
            pltpu.VMEM((bq, D), jnp.float32),
        ],
    )(x, lnv, f1, f1b, f2, f2b)


def kernel(tokens, attention_weights, mask, params):
    aw_c = attention_weights.transpose(1, 3, 0, 2).reshape(BATCH, M, AWR)
    maskf = mask.astype(jnp.float32)[..., None]
    w1t = params['imp_w1'].T
    b1 = params['imp_b1'][None]
    w2t = params['imp_w2'].T
    b2 = params['imp_b2'][None]

    hid_pre = _mlp1(tokens, w1t, b1)
    hid = jax.nn.gelu(hid_pre, approximate=False)
    imp3, comp3 = _importance(hid, aw_c, maskf, w2t, b2)
    importance = imp3[..., 0]

    compr = comp3.reshape(BATCH, 1, M)
    sel3, gidx3 = _topk(compr, comp3)
    selected_indices = sel3[:, 0, :]

    sel_tokens_flat = _sc_gather(tokens.reshape(BATCH * M, D),
                                 gidx3.reshape(BATCH * KSEL))
    sel_tokens = sel_tokens_flat.reshape(BATCH, KSEL, D)

    q = jnp.broadcast_to(
        (params['detail_queries'] + params['detail_pos'])[None],
        (BATCH, NQ, D))

    n_layers = len(params['layers'])
    for li, p in enumerate(params['layers']):
        bq_, bk_, bv_ = jnp.split(p['in_b'], 3)
        vecs = jnp.stack([p['nq_g'], p['nq_b'], p['nkv_g'], p['nkv_b'],
                          bq_, bk_, bv_, p['out_b']])
        q = _attn(q, sel_tokens, vecs, p['in_w'], p['out_w'])
        lnv = jnp.stack([p['nffn_g'], p['nffn_b'],
                         params['on_g'], params['on_b']])
        x = q.reshape(BATCH * NQ, D)
        x = _ffn(x, lnv, p['f1_w'], p['f1_b'][None], p['f2_w'],
                 p['f2_b'][None], final_ln=(li == n_layers - 1))
        q = x.reshape(BATCH, NQ, D)

    return q, importance, selected_indices


# speedup vs baseline: 1.3285x; 1.0030x over previous
import functools

import jax
import jax.numpy as jnp
from jax import lax
from jax.experimental import pallas as pl
from jax.experimental.pallas import tpu as pltpu
from jax.experimental.pallas import tpu_sc as plsc

D = 1024
M = 4096
BATCH = 4
KSEL = 512
NQ = 16
NH = 16
DH = 64
MLPD = 4096
D4 = 256
AWR = 32
MB = 1024
IB = 512
JB = 1024
EPS = 1e-5

_SC_CORES = 2
_SC_SUBCORES = 16
_SC_WORKERS = _SC_CORES * _SC_SUBCORES
_ROWS_PER_W = (BATCH * KSEL) // _SC_WORKERS


def _gelu(x):
    return 0.5 * x * (1.0 + lax.erf(x * 0.7071067811865476))


def _ln(x, g, b):
    mu = jnp.mean(x, axis=-1, keepdims=True)
    xc = x - mu
    var = jnp.mean(xc * xc, axis=-1, keepdims=True)
    return xc / jnp.sqrt(var + EPS) * g + b


def _mlp1_body(tok_ref, w1t_ref, b1_ref, hid_ref):
    tok = tok_ref[0]
    hid_ref[0] = jnp.dot(tok.astype(jnp.bfloat16),
                         w1t_ref[...].astype(jnp.bfloat16),
                         preferred_element_type=jnp.float32) + b1_ref[...]


def _mlp1(tokens, w1t, b1):
    return pl.pallas_call(
        _mlp1_body,
        grid=(BATCH, M // MB),
        in_specs=[
            pl.BlockSpec((1, MB, D), lambda b, m: (b, m, 0)),
            pl.BlockSpec((D, D4), lambda b, m: (0, 0)),
            pl.BlockSpec((1, D4), lambda b, m: (0, 0)),
        ],
        out_specs=pl.BlockSpec((1, MB, D4), lambda b, m: (b, m, 0)),
        out_shape=jax.ShapeDtypeStruct((BATCH, M, D4), jnp.float32),
    )(tokens, w1t, b1)


def _imp_body(hid_ref, aw_ref, maskf_ref, w2t_ref, b2_ref, imp_ref, comp_ref):
    hid = hid_ref[0]
    logit = jnp.dot(hid.astype(jnp.bfloat16), w2t_ref[...].astype(jnp.bfloat16),
                    preferred_element_type=jnp.float32) + b2_ref[...]
    imp = jax.nn.sigmoid(logit)
    cov = jnp.max(aw_ref[0], axis=1, keepdims=True)
    comp = imp * (1.0 - cov)
    comp = jnp.where(maskf_ref[0] > 0.0, comp, -jnp.inf)
    imp_ref[0] = imp
    comp_ref[0] = comp


def _importance(hid, aw_c, maskf, w2t, b2):
    grid = (BATCH, M // MB)
    return pl.pallas_call(
        _imp_body,
        grid=grid,
        in_specs=[
            pl.BlockSpec((1, MB, D4), lambda b, m: (b, m, 0)),
            pl.BlockSpec((1, MB, AWR), lambda b, m: (b, m, 0)),
            pl.BlockSpec((1, MB, 1), lambda b, m: (b, m, 0)),
            pl.BlockSpec((D4, 1), lambda b, m: (0, 0)),
            pl.BlockSpec((1, 1), lambda b, m: (0, 0)),
        ],
        out_specs=[
            pl.BlockSpec((1, MB, 1), lambda b, m: (b, m, 0)),
            pl.BlockSpec((1, MB, 1), lambda b, m: (b, m, 0)),
        ],
        out_shape=[
            jax.ShapeDtypeStruct((BATCH, M, 1), jnp.float32),
            jax.ShapeDtypeStruct((BATCH, M, 1), jnp.float32),
        ],
    )(hid, aw_c, maskf, w2t, b2)


def _topk_body(compr_ref, compc_ref, sel_ref, gidx_ref):
    b = pl.program_id(0)
    vr = compr_ref[0]
    vc = compc_ref[0]
    jj = lax.broadcasted_iota(jnp.int32, (1, M), 1)
    pp = lax.broadcasted_iota(jnp.int32, (1, KSEL), 1)
    sel_acc = jnp.zeros((1, KSEL), jnp.int32)
    for ib in range(M // IB):
        vi = vc[ib * IB:(ib + 1) * IB, :]
        ii = lax.broadcasted_iota(jnp.int32, (IB, 1), 0) + ib * IB
        better = jnp.logical_or(
            vr > vi, jnp.logical_and(vr == vi, jj < ii))
        cnt = jnp.sum(better.astype(jnp.int32), axis=1, keepdims=True)
        match = cnt == pp
        sel_acc = sel_acc + jnp.sum(
            jnp.where(match, ii, 0), axis=0, keepdims=True)
    sel_ref[0] = sel_acc
    gidx_ref[0] = sel_acc + b * M


def _topk(compr, compc):
    return pl.pallas_call(
        _topk_body,
        grid=(BATCH,),
        in_specs=[
            pl.BlockSpec((1, 1, M), lambda b: (b, 0, 0)),
            pl.BlockSpec((1, M, 1), lambda b: (b, 0, 0)),
        ],
        out_specs=[
            pl.BlockSpec((1, 1, KSEL), lambda b: (b, 0, 0)),
            pl.BlockSpec((1, 1, KSEL), lambda b: (b, 0, 0)),
        ],
        out_shape=[
            jax.ShapeDtypeStruct((BATCH, 1, KSEL), jnp.int32),
            jax.ShapeDtypeStruct((BATCH, 1, KSEL), jnp.int32),
        ],
    )(compr, compc)


def _sc_gather(tokens_flat, gidx):
    mesh = plsc.VectorSubcoreMesh(core_axis_name="c", subcore_axis_name="s")

    @functools.partial(
        pl.kernel,
        mesh=mesh,
        out_type=jax.ShapeDtypeStruct((BATCH * KSEL, D), jnp.float32),
        scratch_types=[
            pltpu.VMEM((_ROWS_PER_W,), jnp.int32),
            pltpu.VMEM((_ROWS_PER_W, D), jnp.float32),
            pltpu.SemaphoreType.DMA,
        ],
    )
    def k(tok_hbm, gidx_hbm, out_hbm, idx_v, rows_v, sem):
        wid = lax.axis_index("s") * _SC_CORES + lax.axis_index("c")
        base = wid * _ROWS_PER_W
        pltpu.sync_copy(gidx_hbm.at[pl.ds(base, _ROWS_PER_W)], idx_v)
        pltpu.async_copy(tok_hbm.at[idx_v], rows_v, sem).wait()
        pltpu.sync_copy(rows_v, out_hbm.at[pl.ds(base, _ROWS_PER_W)])

    return k(tokens_flat, gidx)


_NT = (((1,), (1,)), ((), ()))


def _attn_body(q_ref, kv_ref, vecs_ref, inw_ref, wo_ref, q1_ref):
    x = q_ref[...]
    kv = kv_ref[...]
    vecs = vecs_ref[...]
    qn = _ln(x, vecs[0:1], vecs[1:2])
    kvn = _ln(kv, vecs[2:3], vecs[3:4])
    wq = inw_ref[0:D, :]
    wk = inw_ref[D:2 * D, :]
    wv = inw_ref[2 * D:3 * D, :]
    bf = jnp.bfloat16
    qp = lax.dot_general(qn.astype(bf), wq.astype(bf), _NT,
                         preferred_element_type=jnp.float32) + vecs[4:5]
    kp = lax.dot_general(kvn.astype(bf), wk.astype(bf), _NT,
                         preferred_element_type=jnp.float32) + vecs[5:6]
    vp = lax.dot_general(kv.astype(bf), wv.astype(bf), _NT,
                         preferred_element_type=jnp.float32) + vecs[6:7]
    rows = []
    for b in range(BATCH):
        qb = slice(b * NQ, (b + 1) * NQ)
        kb = slice(b * KSEL, (b + 1) * KSEL)
        outs = []
        for h in range(NH):
            sl = slice(h * DH, (h + 1) * DH)
            qh = qp[qb, sl]
            kh = kp[kb, sl]
            vh = vp[kb, sl]
            sc = lax.dot_general(qh.astype(bf), kh.astype(bf), _NT,
                                 preferred_element_type=jnp.float32) * 0.125
            m = jnp.max(sc, axis=1, keepdims=True)
            e = jnp.exp(sc - m)
            a = e / jnp.sum(e, axis=1, keepdims=True)
            outs.append(lax.dot_general(
                a.astype(bf), vh.astype(bf), (((1,), (0,)), ((), ())),
                preferred_element_type=jnp.float32))
        rows.append(jnp.concatenate(outs, axis=1))
    ao = jnp.concatenate(rows, axis=0)
    o = lax.dot_general(ao.astype(bf), wo_ref[...].astype(bf), _NT,
                        preferred_element_type=jnp.float32) + vecs[7:8]
    q1_ref[...] = x + o


def _attn(q, kv, vecs, inw, wo):
    return pl.pallas_call(
        _attn_body,
        in_specs=[
            pl.BlockSpec((BATCH * NQ, D), lambda: (0, 0)),
            pl.BlockSpec((BATCH * KSEL, D), lambda: (0, 0)),
            pl.BlockSpec((8, D), lambda: (0, 0)),
            pl.BlockSpec((3 * D, D), lambda: (0, 0)),
            pl.BlockSpec((D, D), lambda: (0, 0)),
        ],
        out_specs=pl.BlockSpec((BATCH * NQ, D), lambda: (0, 0)),
        out_shape=jax.ShapeDtypeStruct((BATCH * NQ, D), jnp.float32),
    )(q, kv, vecs, inw, wo)


def _ffn_body(x_ref, lnv_ref, f1_ref, f1b_ref, f2_ref, f2b_ref, out_ref,
              hm_s, acc_s, *, final_ln):
    j = pl.program_id(0)

    @pl.when(j == 0)
    def _():
        lnv = lnv_ref[...]
        hm_s[...] = _ln(x_ref[...], lnv[0:1], lnv[1:2])
        acc_s[...] = jnp.zeros_like(acc_s)

    h = lax.dot_general(hm_s[...].astype(jnp.bfloat16),
                        f1_ref[...].astype(jnp.bfloat16), _NT,
                        preferred_element_type=jnp.float32)
    h = _gelu(h + f1b_ref[...])
    acc_s[...] += lax.dot_general(h.astype(jnp.bfloat16),
                                  f2_ref[...].astype(jnp.bfloat16), _NT,
                                  preferred_element_type=jnp.float32)

    @pl.when(j == MLPD // JB - 1)
    def _():
        out = x_ref[...] + acc_s[...] + f2b_ref[...]
        if final_ln:
            lnv = lnv_ref[...]
            out = _ln(out, lnv[2:3], lnv[3:4])
        out_ref[...] = out


def _ffn(x, lnv, f1, f1b, f2, f2b, final_ln):
    bq = BATCH * NQ
    return pl.pallas_call(
        functools.partial(_ffn_body, final_ln=final_ln),
        grid=(MLPD // JB,),
        in_specs=[
            pl.BlockSpec((bq, D), lambda j: (0, 0)),
            pl.BlockSpec((4, D), lambda j: (0, 0)),
            pl.BlockSpec((JB, D), lambda j: (j, 0)),
            pl.BlockSpec((1, JB), lambda j: (0, j)),
            pl.BlockSpec((D, JB), lambda j: (0, j)),
            pl.BlockSpec((1, D), lambda j: (0, 0)),
        ],
        out_specs=pl.BlockSpec((bq, D), lambda j: (0, 0)),
        out_shape=jax.ShapeDtypeStruct((bq, D), jnp.float32),
        scratch_shapes=[
            pltpu.VMEM((bq, D), jnp.float32),
            pltpu.VMEM((bq, D), jnp.float32),
        ],
    )(x, lnv, f1, f1b, f2, f2b)


def kernel(tokens, attention_weights, mask, params):
    aw_c = attention_weights.transpose(1, 3, 0, 2).reshape(BATCH, M, AWR)
    maskf = mask.astype(jnp.float32)[..., None]
    w1t = params['imp_w1'].T
    b1 = params['imp_b1'][None]
    w2t = params['imp_w2'].T
    b2 = params['imp_b2'][None]

    hid_pre = _mlp1(tokens, w1t, b1)
    hid = jax.nn.gelu(hid_pre, approximate=False)
    imp3, comp3 = _importance(hid, aw_c, maskf, w2t, b2)
    importance = imp3[..., 0]

    compr = comp3.reshape(BATCH, 1, M)
    sel3, gidx3 = _topk(compr, comp3)
    selected_indices = sel3[:, 0, :]

    sel_tokens_flat = _sc_gather(tokens.reshape(BATCH * M, D),
                                 gidx3.reshape(BATCH * KSEL))

    x = jnp.broadcast_to(
        (params['detail_queries'] + params['detail_pos'])[None],
        (BATCH, NQ, D)).reshape(BATCH * NQ, D)

    n_layers = len(params['layers'])
    for li, p in enumerate(params['layers']):
        bq_, bk_, bv_ = jnp.split(p['in_b'], 3)
        vecs = jnp.stack([p['nq_g'], p['nq_b'], p['nkv_g'], p['nkv_b'],
                          bq_, bk_, bv_, p['out_b']])
        x = _attn(x, sel_tokens_flat, vecs, p['in_w'], p['out_w'])
        lnv = jnp.stack([p['nffn_g'], p['nffn_b'],
                         params['on_g'], params['on_b']])
        x = _ffn(x, lnv, p['f1_w'], p['f1_b'][None], p['f2_w'],
                 p['f2_b'][None], final_ln=(li == n_layers - 1))

    return x.reshape(BATCH, NQ, D), importance, selected_indices
